# baseline (speedup 1.0000x reference)
import functools

import jax
import jax.numpy as jnp
from jax import lax
from jax.experimental import pallas as pl
from jax.experimental.pallas import tpu as pltpu
from jax.experimental.pallas import tpu_sc as plsc

N = 10000
E = 320000
D = 128
G = 128
NC = 2
NS = 16
NW = NC * NS
DH = D // NC
CHUNK = 128
KPT = 160
KB = 4
NSLAB = KPT // KB
N_PAD = 10112
RPT = N_PAD // NS


_SLICES = tuple((i, min(CHUNK, RPT - i)) for i in range(0, RPT, CHUNK))


def _agg_kernel_body(with_deg, *refs):
    if with_deg:
        (h_hbm, srci, dsti, z_hbm, z16_hbm, o16_hbm,
         out_hbm, dout_hbm, srcv, dstv, rows, rows1, onesv,
         sem, sem1, semd, acc, tbl, dacc) = refs
    else:
        (h_hbm, srci, dsti, z_hbm,
         out_hbm, srcv, dstv, rows, rows1, onesv,
         sem, sem1, semd, acc, tbl, dacc) = refs
    core = lax.axis_index("c")
    sid = lax.axis_index("s")
    r0 = sid * RPT
    c0 = core * DH
    pltpu.sync_copy(z_hbm, rows)
    for off, cnt in _SLICES:
        pltpu.sync_copy(rows.at[pl.ds(0, cnt)], acc.at[pl.ds(r0 + off, cnt)])
    for off, cnt in _SLICES:
        pltpu.sync_copy(h_hbm.at[pl.ds(r0 + off, cnt), pl.ds(c0, DH)],
                        rows.at[pl.ds(0, cnt)])
        pltpu.sync_copy(rows.at[pl.ds(0, cnt)], tbl.at[pl.ds(r0 + off, cnt)])
    if with_deg:
        pltpu.sync_copy(z16_hbm, onesv)
        for off, cnt in _SLICES:
            pltpu.sync_copy(onesv.at[pl.ds(0, cnt)], dacc.at[pl.ds(r0 + off, cnt)])
        pltpu.sync_copy(o16_hbm, onesv)
    plsc.subcore_barrier()

    base = sid * KPT
    half = NSLAB // 2

    @pl.loop(0, NSLAB)
    def _(s):
        pltpu.sync_copy(srci.at[pl.ds(base + s * KB, KB)], srcv)
        pltpu.sync_copy(dsti.at[pl.ds(base + s * KB, KB)], dstv)
        pltpu.async_copy(tbl.at[srcv.at[0]], rows, sem)

        @pl.loop(0, KB // 2 - 1)
        def _(jj):
            k = 2 * jj
            pltpu.async_copy(tbl.at[srcv.at[k + 1]], rows1, sem1)
            pltpu.make_async_copy(z_hbm, rows, sem).wait()
            pltpu.sync_copy(rows, acc.at[dstv.at[k]], add=True)
            pltpu.async_copy(tbl.at[srcv.at[k + 2]], rows, sem)
            pltpu.make_async_copy(z_hbm, rows1, sem1).wait()
            pltpu.sync_copy(rows1, acc.at[dstv.at[k + 1]], add=True)

        pltpu.async_copy(tbl.at[srcv.at[KB - 1]], rows1, sem1)
        pltpu.make_async_copy(z_hbm, rows, sem).wait()
        pltpu.sync_copy(rows, acc.at[dstv.at[KB - 2]], add=True)
        pltpu.make_async_copy(z_hbm, rows1, sem1).wait()
        pltpu.sync_copy(rows1, acc.at[dstv.at[KB - 1]], add=True)

    if with_deg:
        dbase = base + core * (KPT // 2)

        @pl.loop(0, half)
        def _(s):
            pltpu.sync_copy(dsti.at[pl.ds(dbase + s * KB, KB)], dstv)
            for k in range(KB):
                pltpu.async_copy(onesv, dacc.at[dstv.at[k]], semd, add=True)
            for k in range(KB):
                pltpu.make_async_copy(z16_hbm, onesv, semd).wait()

    plsc.subcore_barrier()
    for off, cnt in _SLICES:
        pltpu.sync_copy(acc.at[pl.ds(r0 + off, cnt)], rows.at[pl.ds(0, cnt)])
        pltpu.sync_copy(rows.at[pl.ds(0, cnt)],
                        out_hbm.at[pl.ds(r0 + off, cnt), pl.ds(c0, DH)])
    if with_deg:
        for off, cnt in _SLICES:
            pltpu.sync_copy(dacc.at[pl.ds(r0 + off, cnt)], onesv.at[pl.ds(0, cnt)])
            pltpu.sync_copy(onesv.at[pl.ds(0, cnt)],
                            dout_hbm.at[pl.ds(core * N_PAD + r0 + off, cnt)])


def _make_agg(with_deg):
    mesh = plsc.VectorSubcoreMesh(core_axis_name="c", subcore_axis_name="s")
    if with_deg:
        out_type = (jax.ShapeDtypeStruct((N_PAD, D), jnp.float32),
                    jax.ShapeDtypeStruct((NC * N_PAD, 16), jnp.float32))
    else:
        out_type = jax.ShapeDtypeStruct((N_PAD, D), jnp.float32)
    scratch_types = [
        pltpu.VMEM((KB, CHUNK), jnp.int32),
        pltpu.VMEM((KB, CHUNK), jnp.int32),
        pltpu.VMEM((CHUNK, DH), jnp.float32),
        pltpu.VMEM((CHUNK, DH), jnp.float32),
        pltpu.VMEM((CHUNK, 16), jnp.float32),
        pltpu.SemaphoreType.DMA,
        pltpu.SemaphoreType.DMA,
        pltpu.SemaphoreType.DMA,
        pltpu.VMEM_SHARED((N_PAD, DH), jnp.float32),
        pltpu.VMEM_SHARED((N_PAD, DH), jnp.float32),
        pltpu.VMEM_SHARED((N_PAD, 16), jnp.float32),
    ]
    body = functools.partial(_agg_kernel_body, with_deg)
    return pl.kernel(body, out_type=out_type, mesh=mesh,
                     scratch_types=scratch_types,
                     compiler_params=pltpu.CompilerParams(
                         use_tc_tiling_on_sc=False))


_agg_deg = _make_agg(True)
_agg_nodeg = _make_agg(False)

BLK = 632
NBLK = N_PAD // BLK


def _layer_body(s_ref, d_ref, h_ref, wa_ref, wr_ref, b_ref, o_ref):
    deg = jnp.maximum(d_ref[...], 1.0)
    mean = s_ref[...] / deg
    o_ref[...] = (
        jnp.dot(mean, wa_ref[...], preferred_element_type=jnp.float32,
                precision=lax.Precision.HIGHEST)
        + jnp.dot(h_ref[...], wr_ref[...], preferred_element_type=jnp.float32,
                  precision=lax.Precision.HIGHEST)
        + b_ref[...])


def _layer(sums, deg_col, h, wa, wr, b):
    return pl.pallas_call(
        _layer_body,
        grid=(NBLK,),
        in_specs=[
            pl.BlockSpec((BLK, D), lambda i: (i, 0)),
            pl.BlockSpec((BLK, 1), lambda i: (i, 0)),
            pl.BlockSpec((BLK, D), lambda i: (i, 0)),
            pl.BlockSpec((D, D), lambda i: (0, 0)),
            pl.BlockSpec((D, D), lambda i: (0, 0)),
            pl.BlockSpec((1, D), lambda i: (0, 0)),
        ],
        out_specs=pl.BlockSpec((BLK, D), lambda i: (i, 0)),
        out_shape=jax.ShapeDtypeStruct((N_PAD, D), jnp.float32),
    )(sums, deg_col, h, wa, wr, b)


def _final_body(s_ref, d_ref, h1_ref, wa_ref, wr_ref, b_ref, bat_ref,
                wpa_ref, wpb_ref, bp_ref, o_ref, pa, pb, cnt):
    i = pl.program_id(0)

    @pl.when(i == 0)
    def _():
        pa[...] = jnp.zeros_like(pa)
        pb[...] = jnp.zeros_like(pb)
        cnt[...] = jnp.zeros_like(cnt)

    deg = jnp.maximum(d_ref[...], 1.0)
    mean = s_ref[...] / deg
    h1 = h1_ref[...]
    h2 = (jnp.dot(mean, wa_ref[...], preferred_element_type=jnp.float32,
                  precision=lax.Precision.HIGHEST)
          + jnp.dot(h1, wr_ref[...], preferred_element_type=jnp.float32,
                    precision=lax.Precision.HIGHEST)
          + b_ref[...])
    onehot = (bat_ref[...] == lax.broadcasted_iota(jnp.int32, (BLK, G), 1)
              ).astype(jnp.float32)
    dn = (((0,), (0,)), ((), ()))
    pa[...] += lax.dot_general(onehot, h1, dn,
                               preferred_element_type=jnp.float32,
                               precision=lax.Precision.HIGHEST)
    pb[...] += lax.dot_general(onehot, h2, dn,
                               preferred_element_type=jnp.float32,
                               precision=lax.Precision.HIGHEST)
    cnt[...] += lax.dot_general(onehot, jnp.ones((BLK, 8), jnp.float32), dn,
                                preferred_element_type=jnp.float32,
                                precision=lax.Precision.HIGHEST)

    @pl.when(i == NBLK - 1)
    def _():
        c = jnp.maximum(cnt[:, 0:1], 1.0)
        o_ref[...] = (
            jnp.dot(pa[...] / c, wpa_ref[...], preferred_element_type=jnp.float32,
                    precision=lax.Precision.HIGHEST)
            + jnp.dot(pb[...] / c, wpb_ref[...], preferred_element_type=jnp.float32,
                      precision=lax.Precision.HIGHEST)
            + bp_ref[...])


def _final(sums, deg_col, h1, wa, wr, b, batch2, wpa, wpb, bp_pad):
    return pl.pallas_call(
        _final_body,
        grid=(NBLK,),
        in_specs=[
            pl.BlockSpec((BLK, D), lambda i: (i, 0)),
            pl.BlockSpec((BLK, 1), lambda i: (i, 0)),
            pl.BlockSpec((BLK, D), lambda i: (i, 0)),
            pl.BlockSpec((D, D), lambda i: (0, 0)),
            pl.BlockSpec((D, D), lambda i: (0, 0)),
            pl.BlockSpec((1, D), lambda i: (0, 0)),
            pl.BlockSpec((BLK, 1), lambda i: (i, 0)),
            pl.BlockSpec((D, D), lambda i: (0, 0)),
            pl.BlockSpec((D, D), lambda i: (0, 0)),
            pl.BlockSpec((1, D), lambda i: (0, 0)),
        ],
        out_specs=pl.BlockSpec((G, D), lambda i: (0, 0)),
        out_shape=jax.ShapeDtypeStruct((G, D), jnp.float32),
        scratch_shapes=[
            pltpu.VMEM((G, D), jnp.float32),
            pltpu.VMEM((G, D), jnp.float32),
            pltpu.VMEM((G, 8), jnp.float32),
        ],
    )(sums, deg_col, h1, wa, wr, b, batch2, wpa, wpb, bp_pad)


def kernel(x, pos, edge_index, batch, W0a, b0a, W0r, b0r,
           W1a, b1a, W1r, b1r, Wp, bp):
    h0 = jnp.concatenate((x, pos), axis=1)
    h0p = jnp.concatenate(
        (h0, jnp.zeros((N_PAD - N, D), jnp.float32)), axis=0)

    ei = edge_index.astype(jnp.int32)
    pad = NS * KPT * CHUNK - E
    src2 = jnp.concatenate((ei[0], jnp.full((pad,), N, jnp.int32))
                           ).reshape(NS * KPT, CHUNK)
    dst2 = jnp.concatenate((ei[1], jnp.full((pad,), N, jnp.int32))
                           ).reshape(NS * KPT, CHUNK)
    z = jnp.zeros((CHUNK, DH), jnp.float32)
    z16 = jnp.zeros((CHUNK, 16), jnp.float32)
    o16 = jnp.ones((CHUNK, 16), jnp.float32)

    sums0, dacc = _agg_deg(h0p, src2, dst2, z, z16, o16)
    dacc = dacc.reshape(NC, N_PAD, 16)
    deg_col = (dacc[0, :, 0] + dacc[1, :, 0]).reshape(N_PAD, 1)
    h1 = _layer(sums0, deg_col, h0p, W0a, W0r, (b0a + b0r).reshape(1, D))

    sums1 = _agg_nodeg(h1, src2, dst2, z)

    batch2 = jnp.concatenate(
        (batch.astype(jnp.int32), jnp.full((N_PAD - N,), -1, jnp.int32))
    ).reshape(N_PAD, 1)
    wpa = Wp[:D]
    wpb = Wp[D:]
    pad_w = jnp.zeros((D, D - Wp.shape[1]), jnp.float32)
    wpa = jnp.concatenate((wpa, pad_w), axis=1)
    wpb = jnp.concatenate((wpb, pad_w), axis=1)
    bp_pad = jnp.concatenate((bp, jnp.zeros((D - bp.shape[0],), jnp.float32))
                             ).reshape(1, D)
    out = _final(sums1, deg_col, h1, W1a, W1r,
                 (b1a + b1r).reshape(1, D), batch2, wpa, wpb, bp_pad)
    return out[:, :Wp.shape[1]]


# speedup vs baseline: 1.0493x; 1.0493x over previous
import functools

import jax
import jax.numpy as jnp
from jax import lax
from jax.experimental import pallas as pl
from jax.experimental.pallas import tpu as pltpu
from jax.experimental.pallas import tpu_sc as plsc

N = 10000
E = 320000
D = 128
G = 128
NC = 2
NS = 16
NW = NC * NS
DH = D // NC
CHUNK = 128
KPT = 160
KB = 4
NSLAB = KPT // KB
N_PAD = 10112
RPT = N_PAD // NS


_SLICES = tuple((i, min(CHUNK, RPT - i)) for i in range(0, RPT, CHUNK))


def _agg_kernel_body(with_deg, *refs):
    if with_deg:
        (h_hbm, srci, dsti, z_hbm, z16_hbm, o16_hbm,
         out_hbm, dout_hbm, srcv, dstv, rows, rows1, onesv,
         sem, sem1, semd, acc, tbl, dacc) = refs
    else:
        (h_hbm, srci, dsti, z_hbm,
         out_hbm, srcv, dstv, rows, rows1, onesv,
         sem, sem1, semd, acc, tbl, dacc) = refs
    core = lax.axis_index("c")
    sid = lax.axis_index("s")
    r0 = sid * RPT
    c0 = core * DH
    pltpu.sync_copy(z_hbm, rows)
    for off, cnt in _SLICES:
        pltpu.sync_copy(rows.at[pl.ds(0, cnt)], acc.at[pl.ds(r0 + off, cnt)])
    for off, cnt in _SLICES:
        pltpu.sync_copy(h_hbm.at[pl.ds(r0 + off, cnt), pl.ds(c0, DH)],
                        rows.at[pl.ds(0, cnt)])
        pltpu.sync_copy(rows.at[pl.ds(0, cnt)], tbl.at[pl.ds(r0 + off, cnt)])
    if with_deg:
        pltpu.sync_copy(z16_hbm, onesv)
        for off, cnt in _SLICES:
            pltpu.sync_copy(onesv.at[pl.ds(0, cnt)], dacc.at[pl.ds(r0 + off, cnt)])
        pltpu.sync_copy(o16_hbm, onesv)
    plsc.subcore_barrier()

    base = sid * KPT
    half = NSLAB // 2

    @pl.loop(0, NSLAB)
    def _(s):
        pltpu.sync_copy(srci.at[pl.ds(base + s * KB, KB)], srcv)
        pltpu.sync_copy(dsti.at[pl.ds(base + s * KB, KB)], dstv)
        pltpu.async_copy(tbl.at[srcv.at[0]], rows, sem)

        @pl.loop(0, KB // 2 - 1)
        def _(jj):
            k = 2 * jj
            pltpu.make_async_copy(z_hbm, rows, sem).wait()
            pltpu.async_copy(tbl.at[srcv.at[k + 1]], rows1, sem1)
            pltpu.sync_copy(rows, acc.at[dstv.at[k]], add=True)
            pltpu.make_async_copy(z_hbm, rows1, sem1).wait()
            pltpu.async_copy(tbl.at[srcv.at[k + 2]], rows, sem)
            pltpu.sync_copy(rows1, acc.at[dstv.at[k + 1]], add=True)

        pltpu.make_async_copy(z_hbm, rows, sem).wait()
        pltpu.async_copy(tbl.at[srcv.at[KB - 1]], rows1, sem1)
        pltpu.sync_copy(rows, acc.at[dstv.at[KB - 2]], add=True)
        pltpu.make_async_copy(z_hbm, rows1, sem1).wait()
        pltpu.sync_copy(rows1, acc.at[dstv.at[KB - 1]], add=True)

    if with_deg:
        dbase = base + core * (KPT // 2)

        @pl.loop(0, half)
        def _(s):
            pltpu.sync_copy(dsti.at[pl.ds(dbase + s * KB, KB)], dstv)
            for k in range(KB):
                pltpu.sync_copy(onesv, dacc.at[dstv.at[k]], add=True)

    plsc.subcore_barrier()
    for off, cnt in _SLICES:
        pltpu.sync_copy(acc.at[pl.ds(r0 + off, cnt)], rows.at[pl.ds(0, cnt)])
        pltpu.sync_copy(rows.at[pl.ds(0, cnt)],
                        out_hbm.at[pl.ds(r0 + off, cnt), pl.ds(c0, DH)])
    if with_deg:
        for off, cnt in _SLICES:
            pltpu.sync_copy(dacc.at[pl.ds(r0 + off, cnt)], onesv.at[pl.ds(0, cnt)])
            pltpu.sync_copy(onesv.at[pl.ds(0, cnt)],
                            dout_hbm.at[pl.ds(core * N_PAD + r0 + off, cnt)])


def _make_agg(with_deg):
    mesh = plsc.VectorSubcoreMesh(core_axis_name="c", subcore_axis_name="s")
    if with_deg:
        out_type = (jax.ShapeDtypeStruct((N_PAD, D), jnp.float32),
                    jax.ShapeDtypeStruct((NC * N_PAD, 16), jnp.float32))
    else:
        out_type = jax.ShapeDtypeStruct((N_PAD, D), jnp.float32)
    scratch_types = [
        pltpu.VMEM((KB, CHUNK), jnp.int32),
        pltpu.VMEM((KB, CHUNK), jnp.int32),
        pltpu.VMEM((CHUNK, DH), jnp.float32),
        pltpu.VMEM((CHUNK, DH), jnp.float32),
        pltpu.VMEM((CHUNK, 16), jnp.float32),
        pltpu.SemaphoreType.DMA,
        pltpu.SemaphoreType.DMA,
        pltpu.SemaphoreType.DMA,
        pltpu.VMEM_SHARED((N_PAD, DH), jnp.float32),
        pltpu.VMEM_SHARED((N_PAD, DH), jnp.float32),
        pltpu.VMEM_SHARED((N_PAD, 16), jnp.float32),
    ]
    body = functools.partial(_agg_kernel_body, with_deg)
    return pl.kernel(body, out_type=out_type, mesh=mesh,
                     scratch_types=scratch_types,
                     compiler_params=pltpu.CompilerParams(
                         use_tc_tiling_on_sc=False))


_agg_deg = _make_agg(True)
_agg_nodeg = _make_agg(False)

BLK = 632
NBLK = N_PAD // BLK


def _layer_body(s_ref, d_ref, h_ref, wa_ref, wr_ref, b_ref, o_ref):
    deg = jnp.maximum(d_ref[...], 1.0)
    mean = s_ref[...] / deg
    o_ref[...] = (
        jnp.dot(mean, wa_ref[...], preferred_element_type=jnp.float32,
                precision=lax.Precision.HIGHEST)
        + jnp.dot(h_ref[...], wr_ref[...], preferred_element_type=jnp.float32,
                  precision=lax.Precision.HIGHEST)
        + b_ref[...])


def _layer(sums, deg_col, h, wa, wr, b):
    return pl.pallas_call(
        _layer_body,
        grid=(NBLK,),
        in_specs=[
            pl.BlockSpec((BLK, D), lambda i: (i, 0)),
            pl.BlockSpec((BLK, 1), lambda i: (i, 0)),
            pl.BlockSpec((BLK, D), lambda i: (i, 0)),
            pl.BlockSpec((D, D), lambda i: (0, 0)),
            pl.BlockSpec((D, D), lambda i: (0, 0)),
            pl.BlockSpec((1, D), lambda i: (0, 0)),
        ],
        out_specs=pl.BlockSpec((BLK, D), lambda i: (i, 0)),
        out_shape=jax.ShapeDtypeStruct((N_PAD, D), jnp.float32),
    )(sums, deg_col, h, wa, wr, b)


def _final_body(s_ref, d_ref, h1_ref, wa_ref, wr_ref, b_ref, bat_ref,
                wpa_ref, wpb_ref, bp_ref, o_ref, pa, pb, cnt):
    i = pl.program_id(0)

    @pl.when(i == 0)
    def _():
        pa[...] = jnp.zeros_like(pa)
        pb[...] = jnp.zeros_like(pb)
        cnt[...] = jnp.zeros_like(cnt)

    deg = jnp.maximum(d_ref[...], 1.0)
    mean = s_ref[...] / deg
    h1 = h1_ref[...]
    h2 = (jnp.dot(mean, wa_ref[...], preferred_element_type=jnp.float32,
                  precision=lax.Precision.HIGHEST)
          + jnp.dot(h1, wr_ref[...], preferred_element_type=jnp.float32,
                    precision=lax.Precision.HIGHEST)
          + b_ref[...])
    onehot = (bat_ref[...] == lax.broadcasted_iota(jnp.int32, (BLK, G), 1)
              ).astype(jnp.float32)
    dn = (((0,), (0,)), ((), ()))
    pa[...] += lax.dot_general(onehot, h1, dn,
                               preferred_element_type=jnp.float32,
                               precision=lax.Precision.HIGHEST)
    pb[...] += lax.dot_general(onehot, h2, dn,
                               preferred_element_type=jnp.float32,
                               precision=lax.Precision.HIGHEST)
    cnt[...] += lax.dot_general(onehot, jnp.ones((BLK, 8), jnp.float32), dn,
                                preferred_element_type=jnp.float32,
                                precision=lax.Precision.HIGHEST)

    @pl.when(i == NBLK - 1)
    def _():
        c = jnp.maximum(cnt[:, 0:1], 1.0)
        o_ref[...] = (
            jnp.dot(pa[...] / c, wpa_ref[...], preferred_element_type=jnp.float32,
                    precision=lax.Precision.HIGHEST)
            + jnp.dot(pb[...] / c, wpb_ref[...], preferred_element_type=jnp.float32,
                      precision=lax.Precision.HIGHEST)
            + bp_ref[...])


def _final(sums, deg_col, h1, wa, wr, b, batch2, wpa, wpb, bp_pad):
    return pl.pallas_call(
        _final_body,
        grid=(NBLK,),
        in_specs=[
            pl.BlockSpec((BLK, D), lambda i: (i, 0)),
            pl.BlockSpec((BLK, 1), lambda i: (i, 0)),
            pl.BlockSpec((BLK, D), lambda i: (i, 0)),
            pl.BlockSpec((D, D), lambda i: (0, 0)),
            pl.BlockSpec((D, D), lambda i: (0, 0)),
            pl.BlockSpec((1, D), lambda i: (0, 0)),
            pl.BlockSpec((BLK, 1), lambda i: (i, 0)),
            pl.BlockSpec((D, D), lambda i: (0, 0)),
            pl.BlockSpec((D, D), lambda i: (0, 0)),
            pl.BlockSpec((1, D), lambda i: (0, 0)),
        ],
        out_specs=pl.BlockSpec((G, D), lambda i: (0, 0)),
        out_shape=jax.ShapeDtypeStruct((G, D), jnp.float32),
        scratch_shapes=[
            pltpu.VMEM((G, D), jnp.float32),
            pltpu.VMEM((G, D), jnp.float32),
            pltpu.VMEM((G, 8), jnp.float32),
        ],
    )(sums, deg_col, h1, wa, wr, b, batch2, wpa, wpb, bp_pad)


def kernel(x, pos, edge_index, batch, W0a, b0a, W0r, b0r,
           W1a, b1a, W1r, b1r, Wp, bp):
    h0 = jnp.concatenate((x, pos), axis=1)
    h0p = jnp.concatenate(
        (h0, jnp.zeros((N_PAD - N, D), jnp.float32)), axis=0)

    ei = edge_index.astype(jnp.int32)
    pad = NS * KPT * CHUNK - E
    src2 = jnp.concatenate((ei[0], jnp.full((pad,), N, jnp.int32))
                           ).reshape(NS * KPT, CHUNK)
    dst2 = jnp.concatenate((ei[1], jnp.full((pad,), N, jnp.int32))
                           ).reshape(NS * KPT, CHUNK)
    z = jnp.zeros((CHUNK, DH), jnp.float32)
    z16 = jnp.zeros((CHUNK, 16), jnp.float32)
    o16 = jnp.ones((CHUNK, 16), jnp.float32)

    sums0, dacc = _agg_deg(h0p, src2, dst2, z, z16, o16)
    dacc = dacc.reshape(NC, N_PAD, 16)
    deg_col = (dacc[0, :, 0] + dacc[1, :, 0]).reshape(N_PAD, 1)
    h1 = _layer(sums0, deg_col, h0p, W0a, W0r, (b0a + b0r).reshape(1, D))

    sums1 = _agg_nodeg(h1, src2, dst2, z)

    batch2 = jnp.concatenate(
        (batch.astype(jnp.int32), jnp.full((N_PAD - N,), -1, jnp.int32))
    ).reshape(N_PAD, 1)
    wpa = Wp[:D]
    wpb = Wp[D:]
    pad_w = jnp.zeros((D, D - Wp.shape[1]), jnp.float32)
    wpa = jnp.concatenate((wpa, pad_w), axis=1)
    wpb = jnp.concatenate((wpb, pad_w), axis=1)
    bp_pad = jnp.concatenate((bp, jnp.zeros((D - bp.shape[0],), jnp.float32))
                             ).reshape(1, D)
    out = _final(sums1, deg_col, h1, W1a, W1r,
                 (b1a + b1r).reshape(1, D), batch2, wpa, wpb, bp_pad)
    return out[:, :Wp.shape[1]]


# speedup vs baseline: 1.1422x; 1.0885x over previous
import functools

import jax
import jax.numpy as jnp
from jax import lax
from jax.experimental import pallas as pl
from jax.experimental.pallas import tpu as pltpu
from jax.experimental.pallas import tpu_sc as plsc

N = 10000
E = 320000
D = 128
G = 128
NC = 2
NS = 16
NW = NC * NS
DH = D // NC
CHUNK = 128
KPT = 160
KB = 8
NSLAB = KPT // KB
N_PAD = 10112
RPT = N_PAD // NS


_SLICES = tuple((i, min(CHUNK, RPT - i)) for i in range(0, RPT, CHUNK))


def _agg_kernel_body(with_deg, *refs):
    if with_deg:
        (h_hbm, srci, dsti, z_hbm, z16_hbm, o16_hbm,
         out_hbm, dout_hbm, srcv, dstv, rows, rows1, onesv,
         sem, sem1, semd, acc, tbl, dacc) = refs
    else:
        (h_hbm, srci, dsti, z_hbm,
         out_hbm, srcv, dstv, rows, rows1, onesv,
         sem, sem1, semd, acc, tbl, dacc) = refs
    core = lax.axis_index("c")
    sid = lax.axis_index("s")
    r0 = sid * RPT
    c0 = core * DH
    pltpu.sync_copy(z_hbm, rows)
    for off, cnt in _SLICES:
        pltpu.sync_copy(rows.at[pl.ds(0, cnt)], acc.at[pl.ds(r0 + off, cnt)])
    for off, cnt in _SLICES:
        pltpu.sync_copy(h_hbm.at[pl.ds(r0 + off, cnt), pl.ds(c0, DH)],
                        rows.at[pl.ds(0, cnt)])
        pltpu.sync_copy(rows.at[pl.ds(0, cnt)], tbl.at[pl.ds(r0 + off, cnt)])
    if with_deg:
        pltpu.sync_copy(z16_hbm, onesv)
        for off, cnt in _SLICES:
            pltpu.sync_copy(onesv.at[pl.ds(0, cnt)], dacc.at[pl.ds(r0 + off, cnt)])
        pltpu.sync_copy(o16_hbm, onesv)
    plsc.subcore_barrier()

    base = sid * KPT
    half = NSLAB // 2

    @pl.loop(0, NSLAB)
    def _(s):
        pltpu.sync_copy(srci.at[pl.ds(base + s * KB, KB)], srcv)
        pltpu.sync_copy(dsti.at[pl.ds(base + s * KB, KB)], dstv)
        pltpu.async_copy(tbl.at[srcv.at[0]], rows, sem)

        @pl.loop(0, KB // 2 - 1)
        def _(jj):
            k = 2 * jj
            pltpu.make_async_copy(z_hbm, rows, sem).wait()
            pltpu.async_copy(tbl.at[srcv.at[k + 1]], rows1, sem1)
            pltpu.sync_copy(rows, acc.at[dstv.at[k]], add=True)
            pltpu.make_async_copy(z_hbm, rows1, sem1).wait()
            pltpu.async_copy(tbl.at[srcv.at[k + 2]], rows, sem)
            pltpu.sync_copy(rows1, acc.at[dstv.at[k + 1]], add=True)

        pltpu.make_async_copy(z_hbm, rows, sem).wait()
        pltpu.async_copy(tbl.at[srcv.at[KB - 1]], rows1, sem1)
        pltpu.sync_copy(rows, acc.at[dstv.at[KB - 2]], add=True)
        pltpu.make_async_copy(z_hbm, rows1, sem1).wait()
        pltpu.sync_copy(rows1, acc.at[dstv.at[KB - 1]], add=True)

    if with_deg:
        dbase = base + core * (KPT // 2)

        @pl.loop(0, half)
        def _(s):
            pltpu.sync_copy(dsti.at[pl.ds(dbase + s * KB, KB)], dstv)
            for k in range(KB):
                pltpu.sync_copy(onesv, dacc.at[dstv.at[k]], add=True)

    plsc.subcore_barrier()
    for off, cnt in _SLICES:
        pltpu.sync_copy(acc.at[pl.ds(r0 + off, cnt)], rows.at[pl.ds(0, cnt)])
        pltpu.sync_copy(rows.at[pl.ds(0, cnt)],
                        out_hbm.at[pl.ds(r0 + off, cnt), pl.ds(c0, DH)])
    if with_deg:
        for off, cnt in _SLICES:
            pltpu.sync_copy(dacc.at[pl.ds(r0 + off, cnt)], onesv.at[pl.ds(0, cnt)])
            pltpu.sync_copy(onesv.at[pl.ds(0, cnt)],
                            dout_hbm.at[pl.ds(core * N_PAD + r0 + off, cnt)])


def _make_agg(with_deg):
    mesh = plsc.VectorSubcoreMesh(core_axis_name="c", subcore_axis_name="s")
    if with_deg:
        out_type = (jax.ShapeDtypeStruct((N_PAD, D), jnp.float32),
                    jax.ShapeDtypeStruct((NC * N_PAD, 16), jnp.float32))
    else:
        out_type = jax.ShapeDtypeStruct((N_PAD, D), jnp.float32)
    scratch_types = [
        pltpu.VMEM((KB, CHUNK), jnp.int32),
        pltpu.VMEM((KB, CHUNK), jnp.int32),
        pltpu.VMEM((CHUNK, DH), jnp.float32),
        pltpu.VMEM((CHUNK, DH), jnp.float32),
        pltpu.VMEM((CHUNK, 16), jnp.float32),
        pltpu.SemaphoreType.DMA,
        pltpu.SemaphoreType.DMA,
        pltpu.SemaphoreType.DMA,
        pltpu.VMEM_SHARED((N_PAD, DH), jnp.float32),
        pltpu.VMEM_SHARED((N_PAD, DH), jnp.float32),
        pltpu.VMEM_SHARED((N_PAD, 16), jnp.float32),
    ]
    body = functools.partial(_agg_kernel_body, with_deg)
    return pl.kernel(body, out_type=out_type, mesh=mesh,
                     scratch_types=scratch_types,
                     compiler_params=pltpu.CompilerParams(
                         use_tc_tiling_on_sc=False))


_agg_deg = _make_agg(True)
_agg_nodeg = _make_agg(False)

BLK = 632
NBLK = N_PAD // BLK


def _layer_body(s_ref, d_ref, h_ref, wa_ref, wr_ref, b_ref, o_ref):
    deg = jnp.maximum(d_ref[...], 1.0)
    mean = s_ref[...] / deg
    o_ref[...] = (
        jnp.dot(mean, wa_ref[...], preferred_element_type=jnp.float32,
                precision=lax.Precision.HIGHEST)
        + jnp.dot(h_ref[...], wr_ref[...], preferred_element_type=jnp.float32,
                  precision=lax.Precision.HIGHEST)
        + b_ref[...])


def _layer(sums, deg_col, h, wa, wr, b):
    return pl.pallas_call(
        _layer_body,
        grid=(NBLK,),
        in_specs=[
            pl.BlockSpec((BLK, D), lambda i: (i, 0)),
            pl.BlockSpec((BLK, 1), lambda i: (i, 0)),
            pl.BlockSpec((BLK, D), lambda i: (i, 0)),
            pl.BlockSpec((D, D), lambda i: (0, 0)),
            pl.BlockSpec((D, D), lambda i: (0, 0)),
            pl.BlockSpec((1, D), lambda i: (0, 0)),
        ],
        out_specs=pl.BlockSpec((BLK, D), lambda i: (i, 0)),
        out_shape=jax.ShapeDtypeStruct((N_PAD, D), jnp.float32),
    )(sums, deg_col, h, wa, wr, b)


def _final_body(s_ref, d_ref, h1_ref, wa_ref, wr_ref, b_ref, bat_ref,
                wpa_ref, wpb_ref, bp_ref, o_ref, pa, pb, cnt):
    i = pl.program_id(0)

    @pl.when(i == 0)
    def _():
        pa[...] = jnp.zeros_like(pa)
        pb[...] = jnp.zeros_like(pb)
        cnt[...] = jnp.zeros_like(cnt)

    deg = jnp.maximum(d_ref[...], 1.0)
    mean = s_ref[...] / deg
    h1 = h1_ref[...]
    h2 = (jnp.dot(mean, wa_ref[...], preferred_element_type=jnp.float32,
                  precision=lax.Precision.HIGHEST)
          + jnp.dot(h1, wr_ref[...], preferred_element_type=jnp.float32,
                    precision=lax.Precision.HIGHEST)
          + b_ref[...])
    onehot = (bat_ref[...] == lax.broadcasted_iota(jnp.int32, (BLK, G), 1)
              ).astype(jnp.float32)
    dn = (((0,), (0,)), ((), ()))
    pa[...] += lax.dot_general(onehot, h1, dn,
                               preferred_element_type=jnp.float32,
                               precision=lax.Precision.HIGHEST)
    pb[...] += lax.dot_general(onehot, h2, dn,
                               preferred_element_type=jnp.float32,
                               precision=lax.Precision.HIGHEST)
    cnt[...] += lax.dot_general(onehot, jnp.ones((BLK, 8), jnp.float32), dn,
                                preferred_element_type=jnp.float32,
                                precision=lax.Precision.HIGHEST)

    @pl.when(i == NBLK - 1)
    def _():
        c = jnp.maximum(cnt[:, 0:1], 1.0)
        o_ref[...] = (
            jnp.dot(pa[...] / c, wpa_ref[...], preferred_element_type=jnp.float32,
                    precision=lax.Precision.HIGHEST)
            + jnp.dot(pb[...] / c, wpb_ref[...], preferred_element_type=jnp.float32,
                      precision=lax.Precision.HIGHEST)
            + bp_ref[...])


def _final(sums, deg_col, h1, wa, wr, b, batch2, wpa, wpb, bp_pad):
    return pl.pallas_call(
        _final_body,
        grid=(NBLK,),
        in_specs=[
            pl.BlockSpec((BLK, D), lambda i: (i, 0)),
            pl.BlockSpec((BLK, 1), lambda i: (i, 0)),
            pl.BlockSpec((BLK, D), lambda i: (i, 0)),
            pl.BlockSpec((D, D), lambda i: (0, 0)),
            pl.BlockSpec((D, D), lambda i: (0, 0)),
            pl.BlockSpec((1, D), lambda i: (0, 0)),
            pl.BlockSpec((BLK, 1), lambda i: (i, 0)),
            pl.BlockSpec((D, D), lambda i: (0, 0)),
            pl.BlockSpec((D, D), lambda i: (0, 0)),
            pl.BlockSpec((1, D), lambda i: (0, 0)),
        ],
        out_specs=pl.BlockSpec((G, D), lambda i: (0, 0)),
        out_shape=jax.ShapeDtypeStruct((G, D), jnp.float32),
        scratch_shapes=[
            pltpu.VMEM((G, D), jnp.float32),
            pltpu.VMEM((G, D), jnp.float32),
            pltpu.VMEM((G, 8), jnp.float32),
        ],
    )(sums, deg_col, h1, wa, wr, b, batch2, wpa, wpb, bp_pad)


def kernel(x, pos, edge_index, batch, W0a, b0a, W0r, b0r,
           W1a, b1a, W1r, b1r, Wp, bp):
    h0 = jnp.concatenate((x, pos), axis=1)
    h0p = jnp.concatenate(
        (h0, jnp.zeros((N_PAD - N, D), jnp.float32)), axis=0)

    ei = edge_index.astype(jnp.int32)
    pad = NS * KPT * CHUNK - E
    src2 = jnp.concatenate((ei[0], jnp.full((pad,), N, jnp.int32))
                           ).reshape(NS * KPT, CHUNK)
    dst2 = jnp.concatenate((ei[1], jnp.full((pad,), N, jnp.int32))
                           ).reshape(NS * KPT, CHUNK)
    z = jnp.zeros((CHUNK, DH), jnp.float32)
    z16 = jnp.zeros((CHUNK, 16), jnp.float32)
    o16 = jnp.ones((CHUNK, 16), jnp.float32)

    sums0, dacc = _agg_deg(h0p, src2, dst2, z, z16, o16)
    dacc = dacc.reshape(NC, N_PAD, 16)
    deg_col = (dacc[0, :, 0] + dacc[1, :, 0]).reshape(N_PAD, 1)
    h1 = _layer(sums0, deg_col, h0p, W0a, W0r, (b0a + b0r).reshape(1, D))

    sums1 = _agg_nodeg(h1, src2, dst2, z)

    batch2 = jnp.concatenate(
        (batch.astype(jnp.int32), jnp.full((N_PAD - N,), -1, jnp.int32))
    ).reshape(N_PAD, 1)
    wpa = Wp[:D]
    wpb = Wp[D:]
    pad_w = jnp.zeros((D, D - Wp.shape[1]), jnp.float32)
    wpa = jnp.concatenate((wpa, pad_w), axis=1)
    wpb = jnp.concatenate((wpb, pad_w), axis=1)
    bp_pad = jnp.concatenate((bp, jnp.zeros((D - bp.shape[0],), jnp.float32))
                             ).reshape(1, D)
    out = _final(sums1, deg_col, h1, W1a, W1r,
                 (b1a + b1r).reshape(1, D), batch2, wpa, wpb, bp_pad)
    return out[:, :Wp.shape[1]]
